# baseline (speedup 1.0000x reference)
import functools

import jax
import jax.numpy as jnp
from jax import lax
from jax.experimental import pallas as pl
from jax.experimental.pallas import tpu as pltpu
from jax.experimental.pallas import tpu_sc as plsc

B, N, DIM = 2, 1024, 512
HEADS, DIM_HEAD, M_DIM, NEIGHBORS = 8, 64, 4, 32
INNER = HEADS * DIM_HEAD
SCALE = DIM_HEAD ** -0.5
ROT_DIM = DIM_HEAD // 2
NFREQ = ROT_DIM // 2
KVC = 2 * INNER + 256

MB = 256
RB = 256
NB = 16
NBK = NB * NEIGHBORS

SC_NC, SC_NS = 2, 16
SC_NW = SC_NC * SC_NS
GROWS = B * N * NEIGHBORS
RPW = GROWS // SC_NW
CHUNK = 64
NCHUNK = RPW // CHUNK


def _proj_body(x_ref, c_ref, wq_ref, wkv_ref, q_ref, kvc_ref):
    x = x_ref[...]
    q_ref[...] = jnp.dot(x, wq_ref[...], preferred_element_type=jnp.float32)
    kvc_ref[:, :2 * INNER] = jnp.dot(x, wkv_ref[...],
                                     preferred_element_type=jnp.float32)
    c = c_ref[...]
    chi = c.astype(jnp.bfloat16).astype(jnp.float32)
    clo = c - chi
    z = jnp.zeros((MB, 125), jnp.float32)
    kvc_ref[:, 2 * INNER:] = jnp.concatenate([chi, z, clo, z], axis=1)


def _topk_body(crow_ref, ct_ref, idx_ref):
    cr = crow_ref[0]
    ca = ct_ref[0]
    dx = cr[:, 0:1] - ca[0:1, :]
    dy = cr[:, 1:2] - ca[1:2, :]
    dz = cr[:, 2:3] - ca[2:3, :]
    cur = dx * dx + dy * dy + dz * dz
    iotaf = jax.lax.broadcasted_iota(jnp.int32, (RB, N), 1).astype(jnp.float32)
    lane32 = jax.lax.broadcasted_iota(jnp.int32, (RB, NEIGHBORS), 1)
    acc = jnp.zeros((RB, NEIGHBORS), jnp.float32)
    big = jnp.float32(N)
    for t in range(NEIGHBORS):
        m = jnp.min(cur, axis=1, keepdims=True)
        cand = jnp.where(cur == m, iotaf, big)
        amin = jnp.min(cand, axis=1, keepdims=True)
        acc = jnp.where(lane32 == t, amin, acc)
        cur = jnp.where(iotaf == amin, jnp.inf, cur)
    boff = pl.program_id(0) * N
    idx_ref[...] = (acc.astype(jnp.int32) + boff)[None]


@functools.partial(
    pl.kernel,
    mesh=plsc.VectorSubcoreMesh(core_axis_name="c", subcore_axis_name="s"),
    out_type=jax.ShapeDtypeStruct((GROWS, KVC), jnp.float32),
    scratch_types=[
        pltpu.VMEM((CHUNK,), jnp.int32),
        pltpu.VMEM((CHUNK, KVC), jnp.float32),
        pltpu.SemaphoreType.DMA,
    ],
)
def _sc_gather(table_hbm, idx_hbm, out_hbm, idx_c, rows, sem):
    wid = lax.axis_index("s") * SC_NC + lax.axis_index("c")
    base = wid * RPW

    def body(ci, carry):
        start = base + ci * CHUNK
        pltpu.sync_copy(idx_hbm.at[pl.ds(start, CHUNK)], idx_c)
        pltpu.async_copy(table_hbm.at[idx_c], rows, sem).wait()
        pltpu.sync_copy(rows, out_hbm.at[pl.ds(start, CHUNK)])
        return carry

    lax.fori_loop(0, NCHUNK, body, 0)


def _attn_body(q_ref, sel_ref, cr_ref, hsum_ref, hexp_ref,
               iffr_ref, emat_ref, pmask_ref, wc1_ref, bc1_ref, wc2_ref,
               bc2_ref, lnb_ref, wout_ref, bout_ref, out_ref, cout_ref):
    sel = sel_ref[0]
    k_sel = sel[:, :INNER]
    v_sel = sel[:, INNER:2 * INNER]
    c_sel = (sel[:, 2 * INNER:2 * INNER + 3]
             + sel[:, 2 * INNER + 128:2 * INNER + 131])
    cr = cr_ref[0]
    c_ctr = jnp.broadcast_to(cr[:, None, :], (NB, NEIGHBORS, 3)).reshape(NBK, 3)
    rel = c_ctr - c_sel
    norm = jnp.sqrt(jnp.sum(rel * rel, axis=1, keepdims=True) + 1e-12)

    norm_row = norm.reshape(1, NBK)
    th16t = iffr_ref[...] * norm_row
    c16 = jnp.cos(th16t)
    s16 = jnp.sin(th16t)
    c16h = c16.astype(jnp.bfloat16).astype(jnp.float32)
    s16h = s16.astype(jnp.bfloat16).astype(jnp.float32)
    csin_t = jnp.concatenate([c16h, c16 - c16h, s16h, s16 - s16h], axis=0)
    cs = jax.lax.dot_general(csin_t, emat_ref[...],
                             (((0,), (0,)), ((), ())),
                             preferred_element_type=jnp.float32)
    cth = cs[:, :INNER] + pmask_ref[...]
    sth = cs[:, INNER:]
    lane = jax.lax.broadcasted_iota(jnp.int32, (1, INNER), 1)
    even = (lane % 2) == 0

    def rot(x):
        rl = jnp.concatenate([x[:, 1:], x[:, :1]], axis=1)
        rr = jnp.concatenate([x[:, -1:], x[:, :-1]], axis=1)
        return jnp.where(even, -rl, rr)

    k_rot = k_sel * cth + rot(k_sel) * sth
    v_rot = v_sel * cth + rot(v_sel) * sth

    q = q_ref[0]
    q_rep = jnp.broadcast_to(q[:, None, :], (NB, NEIGHBORS, INNER)).reshape(NBK, INNER)
    qk2 = jnp.dot(q_rep * k_rot, hsum_ref[...],
                  preferred_element_type=jnp.float32) * SCALE

    h = jnp.dot(qk2, wc1_ref[...], preferred_element_type=jnp.float32) + bc1_ref[...]
    h = 0.5 * h * (1.0 + jax.lax.erf(h * (2.0 ** -0.5)))
    cw = jnp.dot(h, wc2_ref[...], preferred_element_type=jnp.float32) + bc2_ref[...]

    normed = rel / jnp.maximum(norm, 1e-8)
    reln = lnb_ref[0, 0] * normed
    wrel = cw * reln
    cout_ref[...] = jnp.sum(wrel.reshape(NB, NEIGHBORS, 3), axis=1)[None]

    qk3 = qk2.reshape(NB, NEIGHBORS, HEADS)
    mx = jnp.max(qk3, axis=1, keepdims=True)
    e = jnp.exp(qk3 - mx)
    attn = e / jnp.sum(e, axis=1, keepdims=True)
    aexp = jnp.dot(attn.reshape(NBK, HEADS), hexp_ref[...],
                   preferred_element_type=jnp.float32)
    osum = jnp.sum((aexp * v_rot).reshape(NB, NEIGHBORS, INNER), axis=1)
    osum2 = jnp.concatenate([osum, osum], axis=1)
    out_ref[...] = (jnp.dot(osum2, wout_ref[...],
                            preferred_element_type=jnp.float32)
                    + bout_ref[...])[None]


def kernel(feats, coors, W_qkv, W_out, b_out, W_c1, b_c1, W_c2, b_c2, ln_w, ln_b):
    f32 = jnp.float32
    x = feats.reshape(B * N, DIM)
    cflat = coors.reshape(B * N, 3)
    Wq = W_qkv[:, :INNER]
    Wkv = W_qkv[:, INNER:]
    q2, kvc = pl.pallas_call(
        _proj_body,
        grid=(B * N // MB,),
        in_specs=[
            pl.BlockSpec((MB, DIM), lambda i: (i, 0)),
            pl.BlockSpec((MB, 3), lambda i: (i, 0)),
            pl.BlockSpec((DIM, INNER), lambda i: (0, 0)),
            pl.BlockSpec((DIM, 2 * INNER), lambda i: (0, 0)),
        ],
        out_specs=[
            pl.BlockSpec((MB, INNER), lambda i: (i, 0)),
            pl.BlockSpec((MB, KVC), lambda i: (i, 0)),
        ],
        out_shape=[
            jax.ShapeDtypeStruct((B * N, INNER), f32),
            jax.ShapeDtypeStruct((B * N, KVC), f32),
        ],
    )(x, cflat, Wq, Wkv)
    q = q2.reshape(B, N, INNER)

    coorsT = jnp.transpose(coors, (0, 2, 1))
    idx = pl.pallas_call(
        _topk_body,
        grid=(B, N // RB),
        in_specs=[
            pl.BlockSpec((1, RB, 3), lambda b, r: (b, r, 0)),
            pl.BlockSpec((1, 3, N), lambda b, r: (b, 0, 0)),
        ],
        out_specs=pl.BlockSpec((1, RB, NEIGHBORS), lambda b, r: (b, r, 0)),
        out_shape=jax.ShapeDtypeStruct((B, N, NEIGHBORS), jnp.int32),
    )(coors, coorsT)
    idxg = idx.reshape(GROWS)

    selg = _sc_gather(kvc, idxg).reshape(B, N * NEIGHBORS, KVC)

    dh = jnp.arange(INNER, dtype=jnp.int32) // DIM_HEAD
    hsum = (dh[:, None] == jnp.arange(HEADS, dtype=jnp.int32)[None, :]).astype(f32)
    hexp = hsum.T
    dm = jnp.arange(INNER, dtype=jnp.int32) % DIM_HEAD
    inv_freq = 1.0 / (10000.0 ** (jnp.arange(0, ROT_DIM, dtype=f32)[::2] / ROT_DIM))
    iffr16 = (100.0 * inv_freq)[:, None]
    emat = ((dm[None, :] < ROT_DIM)
            & ((dm[None, :] // 2) == jnp.arange(NFREQ, dtype=jnp.int32)[:, None])
            ).astype(f32)
    ez = jnp.zeros_like(emat)
    ec = jnp.concatenate([emat, ez], axis=1)
    es = jnp.concatenate([ez, emat], axis=1)
    e4 = jnp.concatenate([ec, ec, es, es], axis=0)
    pmask = (dm >= ROT_DIM).astype(f32)[None, :]
    whi = W_out.astype(jnp.bfloat16).astype(f32)
    w2 = jnp.concatenate([whi, W_out - whi], axis=0)

    out, coors_out = pl.pallas_call(
        _attn_body,
        grid=(B, N // NB),
        in_specs=[
            pl.BlockSpec((1, NB, INNER), lambda b, i: (b, i, 0)),
            pl.BlockSpec((1, NBK, KVC), lambda b, i: (b, i, 0)),
            pl.BlockSpec((1, NB, 3), lambda b, i: (b, i, 0)),
            pl.BlockSpec((INNER, HEADS), lambda b, i: (0, 0)),
            pl.BlockSpec((HEADS, INNER), lambda b, i: (0, 0)),
            pl.BlockSpec((NFREQ, 1), lambda b, i: (0, 0)),
            pl.BlockSpec((4 * NFREQ, 2 * INNER), lambda b, i: (0, 0)),
            pl.BlockSpec((1, INNER), lambda b, i: (0, 0)),
            pl.BlockSpec((HEADS, M_DIM * 4), lambda b, i: (0, 0)),
            pl.BlockSpec((1, M_DIM * 4), lambda b, i: (0, 0)),
            pl.BlockSpec((M_DIM * 4, 1), lambda b, i: (0, 0)),
            pl.BlockSpec((1, 1), lambda b, i: (0, 0)),
            pl.BlockSpec((1, 1), lambda b, i: (0, 0)),
            pl.BlockSpec((2 * INNER, DIM), lambda b, i: (0, 0)),
            pl.BlockSpec((1, DIM), lambda b, i: (0, 0)),
        ],
        out_specs=[
            pl.BlockSpec((1, NB, DIM), lambda b, i: (b, i, 0)),
            pl.BlockSpec((1, NB, 3), lambda b, i: (b, i, 0)),
        ],
        out_shape=[
            jax.ShapeDtypeStruct((B, N, DIM), f32),
            jax.ShapeDtypeStruct((B, N, 3), f32),
        ],
    )(q, selg, coors, hsum, hexp, iffr16, e4, pmask, W_c1,
      b_c1.reshape(1, -1), W_c2, (b_c2 + 0.0).reshape(1, 1),
      ln_b.reshape(1, 1), w2, b_out.reshape(1, -1))
    return out, coors_out


# speedup vs baseline: 1.0313x; 1.0313x over previous
import functools

import jax
import jax.numpy as jnp
from jax import lax
from jax.experimental import pallas as pl
from jax.experimental.pallas import tpu as pltpu
from jax.experimental.pallas import tpu_sc as plsc

B, N, DIM = 2, 1024, 512
HEADS, DIM_HEAD, M_DIM, NEIGHBORS = 8, 64, 4, 32
INNER = HEADS * DIM_HEAD
SCALE = DIM_HEAD ** -0.5
ROT_DIM = DIM_HEAD // 2
NFREQ = ROT_DIM // 2
KVW = 2 * INNER
CPW = 128

MB = 256
RB = 256
NB = 16
NBK = NB * NEIGHBORS

SC_NC, SC_NS = 2, 16
SC_NW = SC_NC * SC_NS
GROWS = B * N * NEIGHBORS
RPW = GROWS // SC_NW
CHUNK = 32
NCHUNK = RPW // CHUNK


def _proj_body(x_ref, c_ref, wq_ref, wkv_ref, q_ref, kv_ref, cp_ref):
    x = x_ref[...]
    q_ref[...] = jnp.dot(x, wq_ref[...], preferred_element_type=jnp.float32)
    kv_ref[...] = jnp.dot(x, wkv_ref[...], preferred_element_type=jnp.float32)
    c = c_ref[...]
    chi = c.astype(jnp.bfloat16).astype(jnp.float32)
    clo = c - chi
    z = jnp.zeros((MB, 61), jnp.float32)
    cp_ref[...] = jnp.concatenate([chi, z, clo, z], axis=1)


def _topk_body(crow_ref, ct_ref, idx_ref):
    cr = crow_ref[0]
    ca = ct_ref[0]
    dx = cr[:, 0:1] - ca[0:1, :]
    dy = cr[:, 1:2] - ca[1:2, :]
    dz = cr[:, 2:3] - ca[2:3, :]
    cur = dx * dx + dy * dy + dz * dz
    iotaf = jax.lax.broadcasted_iota(jnp.int32, (RB, N), 1).astype(jnp.float32)
    lane32 = jax.lax.broadcasted_iota(jnp.int32, (RB, NEIGHBORS), 1)
    acc = jnp.zeros((RB, NEIGHBORS), jnp.float32)
    big = jnp.float32(N)
    for t in range(NEIGHBORS):
        m = jnp.min(cur, axis=1, keepdims=True)
        cand = jnp.where(cur == m, iotaf, big)
        amin = jnp.min(cand, axis=1, keepdims=True)
        acc = jnp.where(lane32 == t, amin, acc)
        cur = jnp.where(iotaf == amin, jnp.inf, cur)
    boff = pl.program_id(0) * N
    idx_ref[...] = (acc.astype(jnp.int32) + boff)[None]


@functools.partial(
    pl.kernel,
    mesh=plsc.VectorSubcoreMesh(core_axis_name="c", subcore_axis_name="s"),
    out_type=(jax.ShapeDtypeStruct((GROWS, KVW), jnp.float32),
              jax.ShapeDtypeStruct((GROWS, CPW), jnp.float32)),
    scratch_types=[
        pltpu.VMEM((CHUNK,), jnp.int32),
        pltpu.VMEM((CHUNK,), jnp.int32),
        pltpu.VMEM((CHUNK, KVW), jnp.float32),
        pltpu.VMEM((CHUNK, KVW), jnp.float32),
        pltpu.VMEM((CHUNK, CPW), jnp.float32),
        pltpu.VMEM((CHUNK, CPW), jnp.float32),
        pltpu.SemaphoreType.DMA,
        pltpu.SemaphoreType.DMA,
    ],
)
def _sc_gather(table_hbm, ctab_hbm, idx_hbm, out_hbm, cout_hbm,
               idx0, idx1, rows0, rows1, crow0, crow1, sem0, sem1):
    wid = lax.axis_index("s") * SC_NC + lax.axis_index("c")
    base = wid * RPW

    def body(p, carry):
        a = base + 2 * p * CHUNK
        bb = a + CHUNK
        pltpu.sync_copy(idx_hbm.at[pl.ds(a, CHUNK)], idx0)
        h0 = pltpu.async_copy(table_hbm.at[idx0], rows0, sem0)
        hc0 = pltpu.async_copy(ctab_hbm.at[idx0], crow0, sem0)
        pltpu.sync_copy(idx_hbm.at[pl.ds(bb, CHUNK)], idx1)
        h1 = pltpu.async_copy(table_hbm.at[idx1], rows1, sem1)
        hc1 = pltpu.async_copy(ctab_hbm.at[idx1], crow1, sem1)
        h0.wait()
        hc0.wait()
        pltpu.sync_copy(rows0, out_hbm.at[pl.ds(a, CHUNK)])
        pltpu.sync_copy(crow0, cout_hbm.at[pl.ds(a, CHUNK)])
        h1.wait()
        hc1.wait()
        pltpu.sync_copy(rows1, out_hbm.at[pl.ds(bb, CHUNK)])
        pltpu.sync_copy(crow1, cout_hbm.at[pl.ds(bb, CHUNK)])
        return carry

    lax.fori_loop(0, NCHUNK // 2, body, 0)


def _attn_body(q_ref, sel_ref, csel_ref, cr_ref, hsum_ref, hexp_ref,
               iffr_ref, emat_ref, pmask_ref, wc1_ref, bc1_ref, wc2_ref,
               bc2_ref, lnb_ref, wout_ref, bout_ref, out_ref, cout_ref):
    sel = sel_ref[0]
    k_sel = sel[:, :INNER]
    v_sel = sel[:, INNER:2 * INNER]
    csel = csel_ref[0]
    c_sel = csel[:, 0:3] + csel[:, 64:67]
    cr = cr_ref[0]
    c_ctr = jnp.broadcast_to(cr[:, None, :], (NB, NEIGHBORS, 3)).reshape(NBK, 3)
    rel = c_ctr - c_sel
    norm = jnp.sqrt(jnp.sum(rel * rel, axis=1, keepdims=True) + 1e-12)

    norm_row = norm.reshape(1, NBK)
    th16t = iffr_ref[...] * norm_row
    c16 = jnp.cos(th16t)
    s16 = jnp.sin(th16t)
    c16h = c16.astype(jnp.bfloat16).astype(jnp.float32)
    s16h = s16.astype(jnp.bfloat16).astype(jnp.float32)
    csin_t = jnp.concatenate([c16h, c16 - c16h, s16h, s16 - s16h], axis=0)
    cs = jax.lax.dot_general(csin_t, emat_ref[...],
                             (((0,), (0,)), ((), ())),
                             preferred_element_type=jnp.float32)
    cth = cs[:, :INNER] + pmask_ref[...]
    sth = cs[:, INNER:]
    lane = jax.lax.broadcasted_iota(jnp.int32, (1, INNER), 1)
    even = (lane % 2) == 0

    def rot(x):
        rl = jnp.concatenate([x[:, 1:], x[:, :1]], axis=1)
        rr = jnp.concatenate([x[:, -1:], x[:, :-1]], axis=1)
        return jnp.where(even, -rl, rr)

    k_rot = k_sel * cth + rot(k_sel) * sth
    v_rot = v_sel * cth + rot(v_sel) * sth

    q = q_ref[0]
    q_rep = jnp.broadcast_to(q[:, None, :], (NB, NEIGHBORS, INNER)).reshape(NBK, INNER)
    qk2 = jnp.dot(q_rep * k_rot, hsum_ref[...],
                  preferred_element_type=jnp.float32) * SCALE

    h = jnp.dot(qk2, wc1_ref[...], preferred_element_type=jnp.float32) + bc1_ref[...]
    h = 0.5 * h * (1.0 + jax.lax.erf(h * (2.0 ** -0.5)))
    cw = jnp.dot(h, wc2_ref[...], preferred_element_type=jnp.float32) + bc2_ref[...]

    normed = rel / jnp.maximum(norm, 1e-8)
    reln = lnb_ref[0, 0] * normed
    wrel = cw * reln
    cout_ref[...] = jnp.sum(wrel.reshape(NB, NEIGHBORS, 3), axis=1)[None]

    qk3 = qk2.reshape(NB, NEIGHBORS, HEADS)
    mx = jnp.max(qk3, axis=1, keepdims=True)
    e = jnp.exp(qk3 - mx)
    attn = e / jnp.sum(e, axis=1, keepdims=True)
    aexp = jnp.dot(attn.reshape(NBK, HEADS), hexp_ref[...],
                   preferred_element_type=jnp.float32)
    osum = jnp.sum((aexp * v_rot).reshape(NB, NEIGHBORS, INNER), axis=1)
    osum2 = jnp.concatenate([osum, osum], axis=1)
    out_ref[...] = (jnp.dot(osum2, wout_ref[...],
                            preferred_element_type=jnp.float32)
                    + bout_ref[...])[None]


def kernel(feats, coors, W_qkv, W_out, b_out, W_c1, b_c1, W_c2, b_c2, ln_w, ln_b):
    f32 = jnp.float32
    x = feats.reshape(B * N, DIM)
    cflat = coors.reshape(B * N, 3)
    Wq = W_qkv[:, :INNER]
    Wkv = W_qkv[:, INNER:]
    q2, kvt, cpt = pl.pallas_call(
        _proj_body,
        grid=(B * N // MB,),
        in_specs=[
            pl.BlockSpec((MB, DIM), lambda i: (i, 0)),
            pl.BlockSpec((MB, 3), lambda i: (i, 0)),
            pl.BlockSpec((DIM, INNER), lambda i: (0, 0)),
            pl.BlockSpec((DIM, 2 * INNER), lambda i: (0, 0)),
        ],
        out_specs=[
            pl.BlockSpec((MB, INNER), lambda i: (i, 0)),
            pl.BlockSpec((MB, KVW), lambda i: (i, 0)),
            pl.BlockSpec((MB, CPW), lambda i: (i, 0)),
        ],
        out_shape=[
            jax.ShapeDtypeStruct((B * N, INNER), f32),
            jax.ShapeDtypeStruct((B * N, KVW), f32),
            jax.ShapeDtypeStruct((B * N, CPW), f32),
        ],
    )(x, cflat, Wq, Wkv)
    q = q2.reshape(B, N, INNER)

    coorsT = jnp.transpose(coors, (0, 2, 1))
    idx = pl.pallas_call(
        _topk_body,
        grid=(B, N // RB),
        in_specs=[
            pl.BlockSpec((1, RB, 3), lambda b, r: (b, r, 0)),
            pl.BlockSpec((1, 3, N), lambda b, r: (b, 0, 0)),
        ],
        out_specs=pl.BlockSpec((1, RB, NEIGHBORS), lambda b, r: (b, r, 0)),
        out_shape=jax.ShapeDtypeStruct((B, N, NEIGHBORS), jnp.int32),
    )(coors, coorsT)
    idxg = idx.reshape(GROWS)

    selg, cselg = _sc_gather(kvt, cpt, idxg)
    selg = selg.reshape(B, N * NEIGHBORS, KVW)
    cselg = cselg.reshape(B, N * NEIGHBORS, CPW)

    dh = jnp.arange(INNER, dtype=jnp.int32) // DIM_HEAD
    hsum = (dh[:, None] == jnp.arange(HEADS, dtype=jnp.int32)[None, :]).astype(f32)
    hexp = hsum.T
    dm = jnp.arange(INNER, dtype=jnp.int32) % DIM_HEAD
    inv_freq = 1.0 / (10000.0 ** (jnp.arange(0, ROT_DIM, dtype=f32)[::2] / ROT_DIM))
    iffr16 = (100.0 * inv_freq)[:, None]
    emat = ((dm[None, :] < ROT_DIM)
            & ((dm[None, :] // 2) == jnp.arange(NFREQ, dtype=jnp.int32)[:, None])
            ).astype(f32)
    ez = jnp.zeros_like(emat)
    ec = jnp.concatenate([emat, ez], axis=1)
    es = jnp.concatenate([ez, emat], axis=1)
    e4 = jnp.concatenate([ec, ec, es, es], axis=0)
    pmask = (dm >= ROT_DIM).astype(f32)[None, :]
    whi = W_out.astype(jnp.bfloat16).astype(f32)
    w2 = jnp.concatenate([whi, W_out - whi], axis=0)

    out, coors_out = pl.pallas_call(
        _attn_body,
        grid=(B, N // NB),
        in_specs=[
            pl.BlockSpec((1, NB, INNER), lambda b, i: (b, i, 0)),
            pl.BlockSpec((1, NBK, KVW), lambda b, i: (b, i, 0)),
            pl.BlockSpec((1, NBK, CPW), lambda b, i: (b, i, 0)),
            pl.BlockSpec((1, NB, 3), lambda b, i: (b, i, 0)),
            pl.BlockSpec((INNER, HEADS), lambda b, i: (0, 0)),
            pl.BlockSpec((HEADS, INNER), lambda b, i: (0, 0)),
            pl.BlockSpec((NFREQ, 1), lambda b, i: (0, 0)),
            pl.BlockSpec((4 * NFREQ, 2 * INNER), lambda b, i: (0, 0)),
            pl.BlockSpec((1, INNER), lambda b, i: (0, 0)),
            pl.BlockSpec((HEADS, M_DIM * 4), lambda b, i: (0, 0)),
            pl.BlockSpec((1, M_DIM * 4), lambda b, i: (0, 0)),
            pl.BlockSpec((M_DIM * 4, 1), lambda b, i: (0, 0)),
            pl.BlockSpec((1, 1), lambda b, i: (0, 0)),
            pl.BlockSpec((1, 1), lambda b, i: (0, 0)),
            pl.BlockSpec((2 * INNER, DIM), lambda b, i: (0, 0)),
            pl.BlockSpec((1, DIM), lambda b, i: (0, 0)),
        ],
        out_specs=[
            pl.BlockSpec((1, NB, DIM), lambda b, i: (b, i, 0)),
            pl.BlockSpec((1, NB, 3), lambda b, i: (b, i, 0)),
        ],
        out_shape=[
            jax.ShapeDtypeStruct((B, N, DIM), f32),
            jax.ShapeDtypeStruct((B, N, 3), f32),
        ],
    )(q, selg, cselg, coors, hsum, hexp, iffr16, e4, pmask, W_c1,
      b_c1.reshape(1, -1), W_c2, (b_c2 + 0.0).reshape(1, 1),
      ln_b.reshape(1, 1), w2, b_out.reshape(1, -1))
    return out, coors_out


# speedup vs baseline: 1.1344x; 1.1000x over previous
import functools

import jax
import jax.numpy as jnp
from jax import lax
from jax.experimental import pallas as pl
from jax.experimental.pallas import tpu as pltpu
from jax.experimental.pallas import tpu_sc as plsc

B, N, DIM = 2, 1024, 512
HEADS, DIM_HEAD, M_DIM, NEIGHBORS = 8, 64, 4, 32
INNER = HEADS * DIM_HEAD
SCALE = DIM_HEAD ** -0.5
ROT_DIM = DIM_HEAD // 2
NFREQ = ROT_DIM // 2
KVW = 2 * INNER
CPW = 128

MB = 256
RB = 256
NB = 16
NBK = NB * NEIGHBORS

SC_NC, SC_NS = 2, 16
SC_NW = SC_NC * SC_NS
GROWS = B * N * NEIGHBORS
GROWS_H = N * NEIGHBORS
RPW = GROWS_H // SC_NW
CHUNK = 32
NCHUNK = RPW // CHUNK


def _proj_body(x_ref, c_ref, wq_ref, wkv_ref, q_ref, kv_ref, cp_ref):
    x = x_ref[...]
    q_ref[...] = jnp.dot(x, wq_ref[...], preferred_element_type=jnp.float32)
    kv_ref[...] = jnp.dot(x, wkv_ref[...], preferred_element_type=jnp.float32)
    c = c_ref[...]
    chi = c.astype(jnp.bfloat16).astype(jnp.float32)
    clo = c - chi
    z = jnp.zeros((MB, 61), jnp.float32)
    cp_ref[...] = jnp.concatenate([chi, z, clo, z], axis=1)


def _topk_body(crow_ref, ct_ref, idx_ref):
    cr = crow_ref[0]
    ca = ct_ref[0]
    dx = cr[:, 0:1] - ca[0:1, :]
    dy = cr[:, 1:2] - ca[1:2, :]
    dz = cr[:, 2:3] - ca[2:3, :]
    cur = dx * dx + dy * dy + dz * dz
    iotaf = jax.lax.broadcasted_iota(jnp.int32, (RB, N), 1).astype(jnp.float32)
    lane32 = jax.lax.broadcasted_iota(jnp.int32, (RB, NEIGHBORS), 1)
    acc = jnp.zeros((RB, NEIGHBORS), jnp.float32)
    big = jnp.float32(N)
    for t in range(NEIGHBORS):
        m = jnp.min(cur, axis=1, keepdims=True)
        cand = jnp.where(cur == m, iotaf, big)
        amin = jnp.min(cand, axis=1, keepdims=True)
        acc = jnp.where(lane32 == t, amin, acc)
        cur = jnp.where(iotaf == amin, jnp.inf, cur)
    boff = pl.program_id(0) * N
    idx_ref[...] = (acc.astype(jnp.int32) + boff)[None]


@functools.partial(
    pl.kernel,
    mesh=plsc.VectorSubcoreMesh(core_axis_name="c", subcore_axis_name="s"),
    out_type=(jax.ShapeDtypeStruct((GROWS_H, KVW), jnp.float32),
              jax.ShapeDtypeStruct((GROWS_H, CPW), jnp.float32)),
    scratch_types=[
        pltpu.VMEM((CHUNK,), jnp.int32),
        pltpu.VMEM((CHUNK,), jnp.int32),
        pltpu.VMEM((CHUNK, KVW), jnp.float32),
        pltpu.VMEM((CHUNK, KVW), jnp.float32),
        pltpu.VMEM((CHUNK, CPW), jnp.float32),
        pltpu.VMEM((CHUNK, CPW), jnp.float32),
        pltpu.SemaphoreType.DMA,
        pltpu.SemaphoreType.DMA,
    ],
)
def _sc_gather(table_hbm, ctab_hbm, idx_hbm, out_hbm, cout_hbm,
               idx0, idx1, rows0, rows1, crow0, crow1, sem0, sem1):
    wid = lax.axis_index("s") * SC_NC + lax.axis_index("c")
    base = wid * RPW

    def body(p, carry):
        a = base + 2 * p * CHUNK
        bb = a + CHUNK
        pltpu.sync_copy(idx_hbm.at[pl.ds(a, CHUNK)], idx0)
        h0 = pltpu.async_copy(table_hbm.at[idx0], rows0, sem0)
        hc0 = pltpu.async_copy(ctab_hbm.at[idx0], crow0, sem0)
        pltpu.sync_copy(idx_hbm.at[pl.ds(bb, CHUNK)], idx1)
        h1 = pltpu.async_copy(table_hbm.at[idx1], rows1, sem1)
        hc1 = pltpu.async_copy(ctab_hbm.at[idx1], crow1, sem1)
        h0.wait()
        hc0.wait()
        pltpu.sync_copy(rows0, out_hbm.at[pl.ds(a, CHUNK)])
        pltpu.sync_copy(crow0, cout_hbm.at[pl.ds(a, CHUNK)])
        h1.wait()
        hc1.wait()
        pltpu.sync_copy(rows1, out_hbm.at[pl.ds(bb, CHUNK)])
        pltpu.sync_copy(crow1, cout_hbm.at[pl.ds(bb, CHUNK)])
        return carry

    lax.fori_loop(0, NCHUNK // 2, body, 0)


def _attn_body(q_ref, sel_ref, csel_ref, cr_ref, hsum_ref, hexp_ref,
               iffr_ref, emat_ref, pmask_ref, wc1_ref, bc1_ref, wc2_ref,
               bc2_ref, lnb_ref, wout_ref, bout_ref, out_ref, cout_ref):
    sel = sel_ref[...]
    k_sel = sel[:, :INNER]
    v_sel = sel[:, INNER:2 * INNER]
    csel = csel_ref[...]
    c_sel = csel[:, 0:3] + csel[:, 64:67]
    cr = cr_ref[...]
    c_ctr = jnp.broadcast_to(cr[:, None, :], (NB, NEIGHBORS, 3)).reshape(NBK, 3)
    rel = c_ctr - c_sel
    norm = jnp.sqrt(jnp.sum(rel * rel, axis=1, keepdims=True) + 1e-12)

    norm_row = norm.reshape(1, NBK)
    th16t = iffr_ref[...] * norm_row
    c16 = jnp.cos(th16t)
    s16 = jnp.sin(th16t)
    c16h = c16.astype(jnp.bfloat16).astype(jnp.float32)
    s16h = s16.astype(jnp.bfloat16).astype(jnp.float32)
    csin_t = jnp.concatenate([c16h, c16 - c16h, s16h, s16 - s16h], axis=0)
    cs = jax.lax.dot_general(csin_t, emat_ref[...],
                             (((0,), (0,)), ((), ())),
                             preferred_element_type=jnp.float32)
    cth = cs[:, :INNER] + pmask_ref[...]
    sth = cs[:, INNER:]
    lane = jax.lax.broadcasted_iota(jnp.int32, (1, INNER), 1)
    even = (lane % 2) == 0

    def rot(x):
        rl = jnp.concatenate([x[:, 1:], x[:, :1]], axis=1)
        rr = jnp.concatenate([x[:, -1:], x[:, :-1]], axis=1)
        return jnp.where(even, -rl, rr)

    k_rot = k_sel * cth + rot(k_sel) * sth
    v_rot = v_sel * cth + rot(v_sel) * sth

    q = q_ref[...]
    q_rep = jnp.broadcast_to(q[:, None, :], (NB, NEIGHBORS, INNER)).reshape(NBK, INNER)
    qk2 = jnp.dot(q_rep * k_rot, hsum_ref[...],
                  preferred_element_type=jnp.float32) * SCALE

    h = jnp.dot(qk2, wc1_ref[...], preferred_element_type=jnp.float32) + bc1_ref[...]
    h = 0.5 * h * (1.0 + jax.lax.erf(h * (2.0 ** -0.5)))
    cw = jnp.dot(h, wc2_ref[...], preferred_element_type=jnp.float32) + bc2_ref[...]

    normed = rel / jnp.maximum(norm, 1e-8)
    reln = lnb_ref[0, 0] * normed
    wrel = cw * reln
    cout_ref[...] = jnp.sum(wrel.reshape(NB, NEIGHBORS, 3), axis=1)

    qk3 = qk2.reshape(NB, NEIGHBORS, HEADS)
    mx = jnp.max(qk3, axis=1, keepdims=True)
    e = jnp.exp(qk3 - mx)
    attn = e / jnp.sum(e, axis=1, keepdims=True)
    aexp = jnp.dot(attn.reshape(NBK, HEADS), hexp_ref[...],
                   preferred_element_type=jnp.float32)
    osum = jnp.sum((aexp * v_rot).reshape(NB, NEIGHBORS, INNER), axis=1)
    osum2 = jnp.concatenate([osum, osum], axis=1)
    out_ref[...] = (jnp.dot(osum2, wout_ref[...],
                            preferred_element_type=jnp.float32)
                    + bout_ref[...])


def kernel(feats, coors, W_qkv, W_out, b_out, W_c1, b_c1, W_c2, b_c2, ln_w, ln_b):
    f32 = jnp.float32
    x = feats.reshape(B * N, DIM)
    cflat = coors.reshape(B * N, 3)
    Wq = W_qkv[:, :INNER]
    Wkv = W_qkv[:, INNER:]
    q2, kvt, cpt = pl.pallas_call(
        _proj_body,
        grid=(B * N // MB,),
        in_specs=[
            pl.BlockSpec((MB, DIM), lambda i: (i, 0)),
            pl.BlockSpec((MB, 3), lambda i: (i, 0)),
            pl.BlockSpec((DIM, INNER), lambda i: (0, 0)),
            pl.BlockSpec((DIM, 2 * INNER), lambda i: (0, 0)),
        ],
        out_specs=[
            pl.BlockSpec((MB, INNER), lambda i: (i, 0)),
            pl.BlockSpec((MB, KVW), lambda i: (i, 0)),
            pl.BlockSpec((MB, CPW), lambda i: (i, 0)),
        ],
        out_shape=[
            jax.ShapeDtypeStruct((B * N, INNER), f32),
            jax.ShapeDtypeStruct((B * N, KVW), f32),
            jax.ShapeDtypeStruct((B * N, CPW), f32),
        ],
    )(x, cflat, Wq, Wkv)
    q = q2.reshape(B, N, INNER)

    coorsT = jnp.transpose(coors, (0, 2, 1))
    idx = pl.pallas_call(
        _topk_body,
        grid=(B, N // RB),
        in_specs=[
            pl.BlockSpec((1, RB, 3), lambda b, r: (b, r, 0)),
            pl.BlockSpec((1, 3, N), lambda b, r: (b, 0, 0)),
        ],
        out_specs=pl.BlockSpec((1, RB, NEIGHBORS), lambda b, r: (b, r, 0)),
        out_shape=jax.ShapeDtypeStruct((B, N, NEIGHBORS), jnp.int32),
    )(coors, coorsT)
    idxg = idx.reshape(B, GROWS_H)

    dh = jnp.arange(INNER, dtype=jnp.int32) // DIM_HEAD
    hsum = (dh[:, None] == jnp.arange(HEADS, dtype=jnp.int32)[None, :]).astype(f32)
    hexp = hsum.T
    dm = jnp.arange(INNER, dtype=jnp.int32) % DIM_HEAD
    inv_freq = 1.0 / (10000.0 ** (jnp.arange(0, ROT_DIM, dtype=f32)[::2] / ROT_DIM))
    iffr16 = (100.0 * inv_freq)[:, None]
    emat = ((dm[None, :] < ROT_DIM)
            & ((dm[None, :] // 2) == jnp.arange(NFREQ, dtype=jnp.int32)[:, None])
            ).astype(f32)
    ez = jnp.zeros_like(emat)
    ec = jnp.concatenate([emat, ez], axis=1)
    es = jnp.concatenate([ez, emat], axis=1)
    e4 = jnp.concatenate([ec, ec, es, es], axis=0)
    pmask = (dm >= ROT_DIM).astype(f32)[None, :]
    whi = W_out.astype(jnp.bfloat16).astype(f32)
    w2 = jnp.concatenate([whi, W_out - whi], axis=0)

    def attn_half(qb, selb, cselb, coorsb):
        return pl.pallas_call(
            _attn_body,
            grid=(N // NB,),
            in_specs=[
                pl.BlockSpec((NB, INNER), lambda i: (i, 0)),
                pl.BlockSpec((NBK, KVW), lambda i: (i, 0)),
                pl.BlockSpec((NBK, CPW), lambda i: (i, 0)),
                pl.BlockSpec((NB, 3), lambda i: (i, 0)),
                pl.BlockSpec((INNER, HEADS), lambda i: (0, 0)),
                pl.BlockSpec((HEADS, INNER), lambda i: (0, 0)),
                pl.BlockSpec((NFREQ, 1), lambda i: (0, 0)),
                pl.BlockSpec((4 * NFREQ, 2 * INNER), lambda i: (0, 0)),
                pl.BlockSpec((1, INNER), lambda i: (0, 0)),
                pl.BlockSpec((HEADS, M_DIM * 4), lambda i: (0, 0)),
                pl.BlockSpec((1, M_DIM * 4), lambda i: (0, 0)),
                pl.BlockSpec((M_DIM * 4, 1), lambda i: (0, 0)),
                pl.BlockSpec((1, 1), lambda i: (0, 0)),
                pl.BlockSpec((1, 1), lambda i: (0, 0)),
                pl.BlockSpec((2 * INNER, DIM), lambda i: (0, 0)),
                pl.BlockSpec((1, DIM), lambda i: (0, 0)),
            ],
            out_specs=[
                pl.BlockSpec((NB, DIM), lambda i: (i, 0)),
                pl.BlockSpec((NB, 3), lambda i: (i, 0)),
            ],
            out_shape=[
                jax.ShapeDtypeStruct((N, DIM), f32),
                jax.ShapeDtypeStruct((N, 3), f32),
            ],
        )(qb, selb, cselb, coorsb, hsum, hexp, iffr16, e4, pmask, W_c1,
          b_c1.reshape(1, -1), W_c2, (b_c2 + 0.0).reshape(1, 1),
          ln_b.reshape(1, 1), w2, b_out.reshape(1, -1))

    outs = []
    couts = []
    sels = [_sc_gather(kvt, cpt, idxg[b]) for b in range(B)]
    for b in range(B):
        selb, cselb = sels[b]
        ob, cb = attn_half(q[b], selb, cselb, coors[b])
        outs.append(ob)
        couts.append(cb)
    return jnp.stack(outs), jnp.stack(couts)


# speedup vs baseline: 1.3248x; 1.1678x over previous
import functools

import jax
import jax.numpy as jnp
from jax import lax
from jax.experimental import pallas as pl
from jax.experimental.pallas import tpu as pltpu
from jax.experimental.pallas import tpu_sc as plsc

B, N, DIM = 2, 1024, 512
HEADS, DIM_HEAD, M_DIM, NEIGHBORS = 8, 64, 4, 32
INNER = HEADS * DIM_HEAD
SCALE = DIM_HEAD ** -0.5
ROT_DIM = DIM_HEAD // 2
NFREQ = ROT_DIM // 2
KVP = INNER
CPW = 128

MB = 256
RB = 256
NB = 16
NBK = NB * NEIGHBORS

SC_NC, SC_NS = 2, 16
SC_NW = SC_NC * SC_NS
GROWS = B * N * NEIGHBORS
GROWS_H = N * NEIGHBORS
RPW = GROWS_H // SC_NW
CHUNK = 64
NCHUNK = RPW // CHUNK


def _bf16_bits(x):
    u = jax.lax.bitcast_convert_type(x, jnp.int32)
    odd = jax.lax.shift_right_logical(u, 16) & 1
    return (u + 0x7FFF + odd) & jnp.int32(-65536)


def _proj_body(x_ref, c_ref, wq_ref, wkv_ref, q_ref, kv_ref, cp_ref):
    x = x_ref[...]
    q_ref[...] = jnp.dot(x, wq_ref[...], preferred_element_type=jnp.float32)
    kv = jnp.dot(x, wkv_ref[...], preferred_element_type=jnp.float32)
    kbits = _bf16_bits(kv[:, :INNER])
    vbits = jax.lax.shift_right_logical(_bf16_bits(kv[:, INNER:]), 16)
    kv_ref[...] = kbits | vbits
    c = c_ref[...]
    chi = c.astype(jnp.bfloat16).astype(jnp.float32)
    clo = c - chi
    z = jnp.zeros((MB, 61), jnp.float32)
    cp_ref[...] = jnp.concatenate([chi, z, clo, z], axis=1)


def _topk_body(crow_ref, ct_ref, idx_ref):
    cr = crow_ref[0]
    ca = ct_ref[0]
    dx = cr[:, 0:1] - ca[0:1, :]
    dy = cr[:, 1:2] - ca[1:2, :]
    dz = cr[:, 2:3] - ca[2:3, :]
    cur = dx * dx + dy * dy + dz * dz
    iotaf = jax.lax.broadcasted_iota(jnp.int32, (RB, N), 1).astype(jnp.float32)
    lane32 = jax.lax.broadcasted_iota(jnp.int32, (RB, NEIGHBORS), 1)
    acc = jnp.zeros((RB, NEIGHBORS), jnp.float32)
    big = jnp.float32(N)
    for t in range(NEIGHBORS):
        m = jnp.min(cur, axis=1, keepdims=True)
        cand = jnp.where(cur == m, iotaf, big)
        amin = jnp.min(cand, axis=1, keepdims=True)
        acc = jnp.where(lane32 == t, amin, acc)
        cur = jnp.where(iotaf == amin, jnp.inf, cur)
    boff = pl.program_id(0) * N
    idx_ref[...] = (acc.astype(jnp.int32) + boff)[None]


@functools.partial(
    pl.kernel,
    mesh=plsc.VectorSubcoreMesh(core_axis_name="c", subcore_axis_name="s"),
    out_type=(jax.ShapeDtypeStruct((GROWS_H, KVP), jnp.int32),
              jax.ShapeDtypeStruct((GROWS_H, CPW), jnp.float32)),
    scratch_types=[
        pltpu.VMEM((CHUNK,), jnp.int32),
        pltpu.VMEM((CHUNK,), jnp.int32),
        pltpu.VMEM((CHUNK, KVP), jnp.int32),
        pltpu.VMEM((CHUNK, KVP), jnp.int32),
        pltpu.VMEM((CHUNK, CPW), jnp.float32),
        pltpu.VMEM((CHUNK, CPW), jnp.float32),
        pltpu.SemaphoreType.DMA,
        pltpu.SemaphoreType.DMA,
    ],
)
def _sc_gather(table_hbm, ctab_hbm, idx_hbm, out_hbm, cout_hbm,
               idx0, idx1, rows0, rows1, crow0, crow1, sem0, sem1):
    wid = lax.axis_index("s") * SC_NC + lax.axis_index("c")
    base = wid * RPW

    def body(p, carry):
        a = base + 2 * p * CHUNK
        bb = a + CHUNK
        pltpu.sync_copy(idx_hbm.at[pl.ds(a, CHUNK)], idx0)
        h0 = pltpu.async_copy(table_hbm.at[idx0], rows0, sem0)
        hc0 = pltpu.async_copy(ctab_hbm.at[idx0], crow0, sem0)
        pltpu.sync_copy(idx_hbm.at[pl.ds(bb, CHUNK)], idx1)
        h1 = pltpu.async_copy(table_hbm.at[idx1], rows1, sem1)
        hc1 = pltpu.async_copy(ctab_hbm.at[idx1], crow1, sem1)
        h0.wait()
        hc0.wait()
        pltpu.sync_copy(rows0, out_hbm.at[pl.ds(a, CHUNK)])
        pltpu.sync_copy(crow0, cout_hbm.at[pl.ds(a, CHUNK)])
        h1.wait()
        hc1.wait()
        pltpu.sync_copy(rows1, out_hbm.at[pl.ds(bb, CHUNK)])
        pltpu.sync_copy(crow1, cout_hbm.at[pl.ds(bb, CHUNK)])
        return carry

    lax.fori_loop(0, NCHUNK // 2, body, 0)


def _attn_body(q_ref, sel_ref, csel_ref, cr_ref, hsum_ref, hexp_ref,
               iffr_ref, emat_ref, pmask_ref, wc1_ref, bc1_ref, wc2_ref,
               bc2_ref, lnb_ref, wout_ref, bout_ref, out_ref, cout_ref):
    sel = sel_ref[...]
    k_sel = jax.lax.bitcast_convert_type(sel & jnp.int32(-65536), jnp.float32)
    v_sel = jax.lax.bitcast_convert_type(
        jax.lax.shift_left(sel, 16), jnp.float32)
    csel = csel_ref[...]
    c_sel = csel[:, 0:3] + csel[:, 64:67]
    cr = cr_ref[...]
    c_ctr = jnp.broadcast_to(cr[:, None, :], (NB, NEIGHBORS, 3)).reshape(NBK, 3)
    rel = c_ctr - c_sel
    norm = jnp.sqrt(jnp.sum(rel * rel, axis=1, keepdims=True) + 1e-12)

    norm_row = norm.reshape(1, NBK)
    th16t = iffr_ref[...] * norm_row
    c16 = jnp.cos(th16t)
    s16 = jnp.sin(th16t)
    c16h = c16.astype(jnp.bfloat16).astype(jnp.float32)
    s16h = s16.astype(jnp.bfloat16).astype(jnp.float32)
    csin_t = jnp.concatenate([c16h, c16 - c16h, s16h, s16 - s16h], axis=0)
    cs = jax.lax.dot_general(csin_t, emat_ref[...],
                             (((0,), (0,)), ((), ())),
                             preferred_element_type=jnp.float32)
    cth = cs[:, :INNER] + pmask_ref[...]
    sth = cs[:, INNER:]
    lane = jax.lax.broadcasted_iota(jnp.int32, (1, INNER), 1)
    even = (lane % 2) == 0

    def rot(x):
        rl = jnp.concatenate([x[:, 1:], x[:, :1]], axis=1)
        rr = jnp.concatenate([x[:, -1:], x[:, :-1]], axis=1)
        return jnp.where(even, -rl, rr)

    k_rot = k_sel * cth + rot(k_sel) * sth
    v_rot = v_sel * cth + rot(v_sel) * sth

    q = q_ref[...]
    q_rep = jnp.broadcast_to(q[:, None, :], (NB, NEIGHBORS, INNER)).reshape(NBK, INNER)
    qk2 = jnp.dot(q_rep * k_rot, hsum_ref[...],
                  preferred_element_type=jnp.float32) * SCALE

    h = jnp.dot(qk2, wc1_ref[...], preferred_element_type=jnp.float32) + bc1_ref[...]
    h = 0.5 * h * (1.0 + jax.lax.erf(h * (2.0 ** -0.5)))
    cw = jnp.dot(h, wc2_ref[...], preferred_element_type=jnp.float32) + bc2_ref[...]

    normed = rel / jnp.maximum(norm, 1e-8)
    reln = lnb_ref[0, 0] * normed
    wrel = cw * reln
    cout_ref[...] = jnp.sum(wrel.reshape(NB, NEIGHBORS, 3), axis=1)

    qk3 = qk2.reshape(NB, NEIGHBORS, HEADS)
    mx = jnp.max(qk3, axis=1, keepdims=True)
    e = jnp.exp(qk3 - mx)
    attn = e / jnp.sum(e, axis=1, keepdims=True)
    aexp = jnp.dot(attn.reshape(NBK, HEADS), hexp_ref[...],
                   preferred_element_type=jnp.float32)
    osum = jnp.sum((aexp * v_rot).reshape(NB, NEIGHBORS, INNER), axis=1)
    osum2 = jnp.concatenate([osum, osum], axis=1)
    out_ref[...] = (jnp.dot(osum2, wout_ref[...],
                            preferred_element_type=jnp.float32)
                    + bout_ref[...])


def kernel(feats, coors, W_qkv, W_out, b_out, W_c1, b_c1, W_c2, b_c2, ln_w, ln_b):
    f32 = jnp.float32
    x = feats.reshape(B * N, DIM)
    cflat = coors.reshape(B * N, 3)
    Wq = W_qkv[:, :INNER]
    Wkv = W_qkv[:, INNER:]
    q2, kvt, cpt = pl.pallas_call(
        _proj_body,
        grid=(B * N // MB,),
        in_specs=[
            pl.BlockSpec((MB, DIM), lambda i: (i, 0)),
            pl.BlockSpec((MB, 3), lambda i: (i, 0)),
            pl.BlockSpec((DIM, INNER), lambda i: (0, 0)),
            pl.BlockSpec((DIM, 2 * INNER), lambda i: (0, 0)),
        ],
        out_specs=[
            pl.BlockSpec((MB, INNER), lambda i: (i, 0)),
            pl.BlockSpec((MB, KVP), lambda i: (i, 0)),
            pl.BlockSpec((MB, CPW), lambda i: (i, 0)),
        ],
        out_shape=[
            jax.ShapeDtypeStruct((B * N, INNER), f32),
            jax.ShapeDtypeStruct((B * N, KVP), jnp.int32),
            jax.ShapeDtypeStruct((B * N, CPW), f32),
        ],
    )(x, cflat, Wq, Wkv)
    q = q2.reshape(B, N, INNER)

    coorsT = jnp.transpose(coors, (0, 2, 1))
    idx = pl.pallas_call(
        _topk_body,
        grid=(B, N // RB),
        in_specs=[
            pl.BlockSpec((1, RB, 3), lambda b, r: (b, r, 0)),
            pl.BlockSpec((1, 3, N), lambda b, r: (b, 0, 0)),
        ],
        out_specs=pl.BlockSpec((1, RB, NEIGHBORS), lambda b, r: (b, r, 0)),
        out_shape=jax.ShapeDtypeStruct((B, N, NEIGHBORS), jnp.int32),
    )(coors, coorsT)
    idxg = idx.reshape(B, GROWS_H)

    dh = jnp.arange(INNER, dtype=jnp.int32) // DIM_HEAD
    hsum = (dh[:, None] == jnp.arange(HEADS, dtype=jnp.int32)[None, :]).astype(f32)
    hexp = hsum.T
    dm = jnp.arange(INNER, dtype=jnp.int32) % DIM_HEAD
    inv_freq = 1.0 / (10000.0 ** (jnp.arange(0, ROT_DIM, dtype=f32)[::2] / ROT_DIM))
    iffr16 = (100.0 * inv_freq)[:, None]
    emat = ((dm[None, :] < ROT_DIM)
            & ((dm[None, :] // 2) == jnp.arange(NFREQ, dtype=jnp.int32)[:, None])
            ).astype(f32)
    ez = jnp.zeros_like(emat)
    ec = jnp.concatenate([emat, ez], axis=1)
    es = jnp.concatenate([ez, emat], axis=1)
    e4 = jnp.concatenate([ec, ec, es, es], axis=0)
    pmask = (dm >= ROT_DIM).astype(f32)[None, :]
    whi = W_out.astype(jnp.bfloat16).astype(f32)
    w2 = jnp.concatenate([whi, W_out - whi], axis=0)

    def attn_half(qb, selb, cselb, coorsb):
        return pl.pallas_call(
            _attn_body,
            grid=(N // NB,),
            in_specs=[
                pl.BlockSpec((NB, INNER), lambda i: (i, 0)),
                pl.BlockSpec((NBK, KVP), lambda i: (i, 0)),
                pl.BlockSpec((NBK, CPW), lambda i: (i, 0)),
                pl.BlockSpec((NB, 3), lambda i: (i, 0)),
                pl.BlockSpec((INNER, HEADS), lambda i: (0, 0)),
                pl.BlockSpec((HEADS, INNER), lambda i: (0, 0)),
                pl.BlockSpec((NFREQ, 1), lambda i: (0, 0)),
                pl.BlockSpec((4 * NFREQ, 2 * INNER), lambda i: (0, 0)),
                pl.BlockSpec((1, INNER), lambda i: (0, 0)),
                pl.BlockSpec((HEADS, M_DIM * 4), lambda i: (0, 0)),
                pl.BlockSpec((1, M_DIM * 4), lambda i: (0, 0)),
                pl.BlockSpec((M_DIM * 4, 1), lambda i: (0, 0)),
                pl.BlockSpec((1, 1), lambda i: (0, 0)),
                pl.BlockSpec((1, 1), lambda i: (0, 0)),
                pl.BlockSpec((2 * INNER, DIM), lambda i: (0, 0)),
                pl.BlockSpec((1, DIM), lambda i: (0, 0)),
            ],
            out_specs=[
                pl.BlockSpec((NB, DIM), lambda i: (i, 0)),
                pl.BlockSpec((NB, 3), lambda i: (i, 0)),
            ],
            out_shape=[
                jax.ShapeDtypeStruct((N, DIM), f32),
                jax.ShapeDtypeStruct((N, 3), f32),
            ],
        )(qb, selb, cselb, coorsb, hsum, hexp, iffr16, e4, pmask, W_c1,
          b_c1.reshape(1, -1), W_c2, (b_c2 + 0.0).reshape(1, 1),
          ln_b.reshape(1, 1), w2, b_out.reshape(1, -1))

    outs = []
    couts = []
    sels = [_sc_gather(kvt, cpt, idxg[b]) for b in range(B)]
    for b in range(B):
        selb, cselb = sels[b]
        ob, cb = attn_half(q[b], selb, cselb, coors[b])
        outs.append(ob)
        couts.append(cb)
    return jnp.stack(outs), jnp.stack(couts)


# speedup vs baseline: 1.4807x; 1.1177x over previous
import functools

import jax
import jax.numpy as jnp
from jax import lax
from jax.experimental import pallas as pl
from jax.experimental.pallas import tpu as pltpu
from jax.experimental.pallas import tpu_sc as plsc

B, N, DIM = 2, 1024, 512
HEADS, DIM_HEAD, M_DIM, NEIGHBORS = 8, 64, 4, 32
INNER = HEADS * DIM_HEAD
SCALE = DIM_HEAD ** -0.5
ROT_DIM = DIM_HEAD // 2
NFREQ = ROT_DIM // 2
KVP = INNER
CPW = 128

MB = 256
RB = 256
NB = 32
NBK = NB * NEIGHBORS

SC_NC, SC_NS = 2, 16
SC_NW = SC_NC * SC_NS
GROWS = B * N * NEIGHBORS
GROWS_H = N * NEIGHBORS
RPW = GROWS_H // SC_NW
CHUNK = 64
NCHUNK = RPW // CHUNK


def _bf16_bits(x):
    u = jax.lax.bitcast_convert_type(x, jnp.int32)
    odd = jax.lax.shift_right_logical(u, 16) & 1
    return (u + 0x7FFF + odd) & jnp.int32(-65536)


def _proj_body(x_ref, c_ref, wq_ref, wkv_ref, q_ref, kv_ref, cp_ref):
    x = x_ref[...]
    q_ref[...] = jnp.dot(x, wq_ref[...], preferred_element_type=jnp.float32)
    kv = jnp.dot(x, wkv_ref[...], preferred_element_type=jnp.float32)
    kbits = _bf16_bits(kv[:, :INNER])
    vbits = jax.lax.shift_right_logical(_bf16_bits(kv[:, INNER:]), 16)
    kv_ref[...] = kbits | vbits
    c = c_ref[...]
    chi = c.astype(jnp.bfloat16).astype(jnp.float32)
    clo = c - chi
    z = jnp.zeros((MB, 61), jnp.float32)
    cp_ref[...] = jnp.concatenate([chi, z, clo, z], axis=1)


def _topk_body(boff, crow_ref, ct_ref, idx_ref):
    cr = crow_ref[...]
    ca = ct_ref[...]
    dx = cr[:, 0:1] - ca[0:1, :]
    dy = cr[:, 1:2] - ca[1:2, :]
    dz = cr[:, 2:3] - ca[2:3, :]
    cur = dx * dx + dy * dy + dz * dz
    iotaf = jax.lax.broadcasted_iota(jnp.int32, (RB, N), 1).astype(jnp.float32)
    lane32 = jax.lax.broadcasted_iota(jnp.int32, (RB, NEIGHBORS), 1)
    acc = jnp.zeros((RB, NEIGHBORS), jnp.float32)
    big = jnp.float32(N)
    for t in range(NEIGHBORS):
        m = jnp.min(cur, axis=1, keepdims=True)
        cand = jnp.where(cur == m, iotaf, big)
        amin = jnp.min(cand, axis=1, keepdims=True)
        acc = jnp.where(lane32 == t, amin, acc)
        cur = jnp.where(iotaf == amin, jnp.inf, cur)
    idx_ref[...] = acc.astype(jnp.int32) + boff


@functools.partial(
    pl.kernel,
    mesh=plsc.VectorSubcoreMesh(core_axis_name="c", subcore_axis_name="s"),
    out_type=(jax.ShapeDtypeStruct((GROWS_H, KVP), jnp.int32),
              jax.ShapeDtypeStruct((GROWS_H, CPW), jnp.float32)),
    scratch_types=[
        pltpu.VMEM((CHUNK,), jnp.int32),
        pltpu.VMEM((CHUNK,), jnp.int32),
        pltpu.VMEM((CHUNK, KVP), jnp.int32),
        pltpu.VMEM((CHUNK, KVP), jnp.int32),
        pltpu.VMEM((CHUNK, CPW), jnp.float32),
        pltpu.VMEM((CHUNK, CPW), jnp.float32),
        pltpu.SemaphoreType.DMA,
        pltpu.SemaphoreType.DMA,
    ],
)
def _sc_gather(table_hbm, ctab_hbm, idx_hbm, out_hbm, cout_hbm,
               idx0, idx1, rows0, rows1, crow0, crow1, sem0, sem1):
    wid = lax.axis_index("s") * SC_NC + lax.axis_index("c")
    base = wid * RPW

    def body(p, carry):
        a = base + 2 * p * CHUNK
        bb = a + CHUNK
        pltpu.sync_copy(idx_hbm.at[pl.ds(a, CHUNK)], idx0)
        h0 = pltpu.async_copy(table_hbm.at[idx0], rows0, sem0)
        hc0 = pltpu.async_copy(ctab_hbm.at[idx0], crow0, sem0)
        pltpu.sync_copy(idx_hbm.at[pl.ds(bb, CHUNK)], idx1)
        h1 = pltpu.async_copy(table_hbm.at[idx1], rows1, sem1)
        hc1 = pltpu.async_copy(ctab_hbm.at[idx1], crow1, sem1)
        h0.wait()
        hc0.wait()
        pltpu.sync_copy(rows0, out_hbm.at[pl.ds(a, CHUNK)])
        pltpu.sync_copy(crow0, cout_hbm.at[pl.ds(a, CHUNK)])
        h1.wait()
        hc1.wait()
        pltpu.sync_copy(rows1, out_hbm.at[pl.ds(bb, CHUNK)])
        pltpu.sync_copy(crow1, cout_hbm.at[pl.ds(bb, CHUNK)])
        return carry

    lax.fori_loop(0, NCHUNK // 2, body, 0)


def _attn_body(q_ref, sel_ref, csel_ref, cr_ref, hsum_ref, hexp_ref,
               iffr_ref, emat_ref, pmask_ref, wc1_ref, bc1_ref, wc2_ref,
               bc2_ref, lnb_ref, wout_ref, bout_ref, out_ref, cout_ref):
    sel = sel_ref[...]
    k_sel = jax.lax.bitcast_convert_type(sel & jnp.int32(-65536), jnp.float32)
    v_sel = jax.lax.bitcast_convert_type(
        jax.lax.shift_left(sel, 16), jnp.float32)
    csel = csel_ref[...]
    c_sel = csel[:, 0:3] + csel[:, 64:67]
    cr = cr_ref[...]
    c_ctr = jnp.broadcast_to(cr[:, None, :], (NB, NEIGHBORS, 3)).reshape(NBK, 3)
    rel = c_ctr - c_sel
    norm = jnp.sqrt(jnp.sum(rel * rel, axis=1, keepdims=True) + 1e-12)

    norm_row = norm.reshape(1, NBK)
    th16t = iffr_ref[...] * norm_row
    c16 = jnp.cos(th16t)
    s16 = jnp.sin(th16t)
    c16h = c16.astype(jnp.bfloat16).astype(jnp.float32)
    s16h = s16.astype(jnp.bfloat16).astype(jnp.float32)
    csin_t = jnp.concatenate([c16h, c16 - c16h, s16h, s16 - s16h], axis=0)
    cs = jax.lax.dot_general(csin_t, emat_ref[...],
                             (((0,), (0,)), ((), ())),
                             preferred_element_type=jnp.float32)
    cth = cs[:, :INNER] + pmask_ref[...]
    sth = cs[:, INNER:]
    lane = jax.lax.broadcasted_iota(jnp.int32, (1, INNER), 1)
    even = (lane % 2) == 0

    def rot(x):
        rl = jnp.concatenate([x[:, 1:], x[:, :1]], axis=1)
        rr = jnp.concatenate([x[:, -1:], x[:, :-1]], axis=1)
        return jnp.where(even, -rl, rr)

    k_rot = k_sel * cth + rot(k_sel) * sth
    v_rot = v_sel * cth + rot(v_sel) * sth

    q = q_ref[...]
    q_rep = jnp.broadcast_to(q[:, None, :], (NB, NEIGHBORS, INNER)).reshape(NBK, INNER)
    qk2 = jnp.dot(q_rep * k_rot, hsum_ref[...],
                  preferred_element_type=jnp.float32) * SCALE

    h = jnp.dot(qk2, wc1_ref[...], preferred_element_type=jnp.float32) + bc1_ref[...]
    h = 0.5 * h * (1.0 + jax.lax.erf(h * (2.0 ** -0.5)))
    cw = jnp.dot(h, wc2_ref[...], preferred_element_type=jnp.float32) + bc2_ref[...]

    normed = rel / jnp.maximum(norm, 1e-8)
    reln = lnb_ref[0, 0] * normed
    wrel = cw * reln
    cout_ref[...] = jnp.sum(wrel.reshape(NB, NEIGHBORS, 3), axis=1)

    qk3 = qk2.reshape(NB, NEIGHBORS, HEADS)
    mx = jnp.max(qk3, axis=1, keepdims=True)
    e = jnp.exp(qk3 - mx)
    attn = e / jnp.sum(e, axis=1, keepdims=True)
    aexp = jnp.dot(attn.reshape(NBK, HEADS), hexp_ref[...],
                   preferred_element_type=jnp.float32)
    osum = jnp.sum((aexp * v_rot).reshape(NB, NEIGHBORS, INNER), axis=1)
    osum2 = jnp.concatenate([osum, osum], axis=1)
    out_ref[...] = (jnp.dot(osum2, wout_ref[...],
                            preferred_element_type=jnp.float32)
                    + bout_ref[...])


def kernel(feats, coors, W_qkv, W_out, b_out, W_c1, b_c1, W_c2, b_c2, ln_w, ln_b):
    f32 = jnp.float32
    x = feats.reshape(B * N, DIM)
    cflat = coors.reshape(B * N, 3)
    Wq = W_qkv[:, :INNER]
    Wkv = W_qkv[:, INNER:]
    q2, kvt, cpt = pl.pallas_call(
        _proj_body,
        grid=(B * N // MB,),
        in_specs=[
            pl.BlockSpec((MB, DIM), lambda i: (i, 0)),
            pl.BlockSpec((MB, 3), lambda i: (i, 0)),
            pl.BlockSpec((DIM, INNER), lambda i: (0, 0)),
            pl.BlockSpec((DIM, 2 * INNER), lambda i: (0, 0)),
        ],
        out_specs=[
            pl.BlockSpec((MB, INNER), lambda i: (i, 0)),
            pl.BlockSpec((MB, KVP), lambda i: (i, 0)),
            pl.BlockSpec((MB, CPW), lambda i: (i, 0)),
        ],
        out_shape=[
            jax.ShapeDtypeStruct((B * N, INNER), f32),
            jax.ShapeDtypeStruct((B * N, KVP), jnp.int32),
            jax.ShapeDtypeStruct((B * N, CPW), f32),
        ],
    )(x, cflat, Wq, Wkv)
    q = q2.reshape(B, N, INNER)

    coorsT = jnp.transpose(coors, (0, 2, 1))

    def topk_half(b):
        return pl.pallas_call(
            functools.partial(_topk_body, b * N),
            grid=(N // RB,),
            in_specs=[
                pl.BlockSpec((RB, 3), lambda r: (r, 0)),
                pl.BlockSpec((3, N), lambda r: (0, 0)),
            ],
            out_specs=pl.BlockSpec((RB, NEIGHBORS), lambda r: (r, 0)),
            out_shape=jax.ShapeDtypeStruct((N, NEIGHBORS), jnp.int32),
        )(coors[b], coorsT[b]).reshape(GROWS_H)

    idxg = [topk_half(b) for b in range(B)]

    dh = jnp.arange(INNER, dtype=jnp.int32) // DIM_HEAD
    hsum = (dh[:, None] == jnp.arange(HEADS, dtype=jnp.int32)[None, :]).astype(f32)
    hexp = hsum.T
    dm = jnp.arange(INNER, dtype=jnp.int32) % DIM_HEAD
    inv_freq = 1.0 / (10000.0 ** (jnp.arange(0, ROT_DIM, dtype=f32)[::2] / ROT_DIM))
    iffr16 = (100.0 * inv_freq)[:, None]
    emat = ((dm[None, :] < ROT_DIM)
            & ((dm[None, :] // 2) == jnp.arange(NFREQ, dtype=jnp.int32)[:, None])
            ).astype(f32)
    ez = jnp.zeros_like(emat)
    ec = jnp.concatenate([emat, ez], axis=1)
    es = jnp.concatenate([ez, emat], axis=1)
    e4 = jnp.concatenate([ec, ec, es, es], axis=0)
    pmask = (dm >= ROT_DIM).astype(f32)[None, :]
    whi = W_out.astype(jnp.bfloat16).astype(f32)
    w2 = jnp.concatenate([whi, W_out - whi], axis=0)

    def attn_half(qb, selb, cselb, coorsb):
        return pl.pallas_call(
            _attn_body,
            grid=(N // NB,),
            in_specs=[
                pl.BlockSpec((NB, INNER), lambda i: (i, 0)),
                pl.BlockSpec((NBK, KVP), lambda i: (i, 0)),
                pl.BlockSpec((NBK, CPW), lambda i: (i, 0)),
                pl.BlockSpec((NB, 3), lambda i: (i, 0)),
                pl.BlockSpec((INNER, HEADS), lambda i: (0, 0)),
                pl.BlockSpec((HEADS, INNER), lambda i: (0, 0)),
                pl.BlockSpec((NFREQ, 1), lambda i: (0, 0)),
                pl.BlockSpec((4 * NFREQ, 2 * INNER), lambda i: (0, 0)),
                pl.BlockSpec((1, INNER), lambda i: (0, 0)),
                pl.BlockSpec((HEADS, M_DIM * 4), lambda i: (0, 0)),
                pl.BlockSpec((1, M_DIM * 4), lambda i: (0, 0)),
                pl.BlockSpec((M_DIM * 4, 1), lambda i: (0, 0)),
                pl.BlockSpec((1, 1), lambda i: (0, 0)),
                pl.BlockSpec((1, 1), lambda i: (0, 0)),
                pl.BlockSpec((2 * INNER, DIM), lambda i: (0, 0)),
                pl.BlockSpec((1, DIM), lambda i: (0, 0)),
            ],
            out_specs=[
                pl.BlockSpec((NB, DIM), lambda i: (i, 0)),
                pl.BlockSpec((NB, 3), lambda i: (i, 0)),
            ],
            out_shape=[
                jax.ShapeDtypeStruct((N, DIM), f32),
                jax.ShapeDtypeStruct((N, 3), f32),
            ],
        )(qb, selb, cselb, coorsb, hsum, hexp, iffr16, e4, pmask, W_c1,
          b_c1.reshape(1, -1), W_c2, (b_c2 + 0.0).reshape(1, 1),
          ln_b.reshape(1, 1), w2, b_out.reshape(1, -1))

    outs = []
    couts = []
    sels = [_sc_gather(kvt, cpt, idxg[b]) for b in range(B)]
    for b in range(B):
        selb, cselb = sels[b]
        ob, cb = attn_half(q[b], selb, cselb, coors[b])
        outs.append(ob)
        couts.append(cb)
    return jnp.stack(outs), jnp.stack(couts)
